# baseline (speedup 1.0000x reference)
import functools

import jax
import jax.numpy as jnp
from jax import lax
from jax.experimental import pallas as pl
from jax.experimental.pallas import tpu as pltpu
from jax.experimental.pallas import tpu_sc as plsc


@functools.cache
def _build(B, D, V):
    info = plsc.get_sparse_core_info()
    NC, NS, L = info.num_cores, info.num_subcores, info.num_lanes
    NW = NC * NS
    assert B % NW == 0 and D == 64 and V % 2 == 0
    bpw = B // NW
    C = 128
    NCHUNK = bpw // C
    mesh = plsc.VectorSubcoreMesh(core_axis_name="c", subcore_axis_name="s")

    @functools.partial(
        pl.kernel,
        mesh=mesh,
        out_type=(
            jax.ShapeDtypeStruct((B // 2, 2 * D), jnp.float32),
            jax.ShapeDtypeStruct((B // 2, 2 * D), jnp.float32),
        ),
        scratch_types=[
            pltpu.VMEM((bpw,), jnp.int32),
            pltpu.VMEM((bpw,), jnp.int32),
            pltpu.VMEM((C, 2 * D), jnp.float32),
            pltpu.VMEM((bpw // 2, 2 * D), jnp.float32),
            pltpu.SemaphoreType.DMA,
        ],
    )
    def k(uid, pid, ut2, pt2, u_out, p_out, ids_v, pidx_v, pair_v, rows_v, sem):
        wid = lax.axis_index("s") * NC + lax.axis_index("c")
        base = pl.multiple_of(wid * bpw, bpw)

        def one_table(idx_hbm, tab2, out_hbm):
            pltpu.sync_copy(idx_hbm.at[pl.ds(base, bpw)], ids_v)

            def shift_body(g, carry):
                v = ids_v[pl.ds(g * L, L)]
                pidx_v[pl.ds(g * L, L)] = lax.shift_right_logical(v, 1)
                return carry

            lax.fori_loop(0, bpw // L, shift_body, 0)

            def chunk_body(c, carry):
                pltpu.async_copy(
                    tab2.at[pidx_v.at[pl.ds(c * C, C)]], pair_v, sem).wait()
                for g in range(C // L):
                    ids_vec = ids_v[pl.ds(c * C + g * L, L)]
                    for l in range(L):
                        off = (ids_vec[l] & 1) * D
                        i = g * L + l
                        dst_row = c * (C // 2) + i // 2
                        dst_off = (i % 2) * D
                        for q in range(D // L):
                            rows_v[dst_row, pl.ds(dst_off + q * L, L)] = (
                                pair_v[i, pl.ds(off + q * L, L)])
                return carry

            lax.fori_loop(0, NCHUNK, chunk_body, 0)
            obase = pl.multiple_of(wid * (bpw // 2), bpw // 2)
            pltpu.sync_copy(rows_v, out_hbm.at[pl.ds(obase, bpw // 2)])

        one_table(uid, ut2, u_out)
        one_table(pid, pt2, p_out)

    return k


def kernel(user_ids, product_ids, user_table, product_table):
    V, D = user_table.shape
    B = user_ids.shape[0]
    k = _build(B, D, V)
    ut2 = user_table.reshape(V // 2, 2 * D)
    pt2 = product_table.reshape(V // 2, 2 * D)
    u2, p2 = k(user_ids.astype(jnp.int32), product_ids.astype(jnp.int32),
               ut2, pt2)
    return u2.reshape(B, D), p2.reshape(B, D)


# speedup vs baseline: 2.2505x; 2.2505x over previous
import functools

import jax
import jax.numpy as jnp
from jax import lax
from jax.experimental import pallas as pl
from jax.experimental.pallas import tpu as pltpu
from jax.experimental.pallas import tpu_sc as plsc


@functools.cache
def _build(B, D, V):
    info = plsc.get_sparse_core_info()
    NC, NS, L = info.num_cores, info.num_subcores, info.num_lanes
    NW = NC * NS
    assert B % (L * NW) == 0 and D % L == 0
    bpw = B // NW
    NG = bpw // L
    NBUF = 8
    SUB = 4
    NSUB = L // SUB
    NTC_FULL = (V // 128) * 128
    mesh = plsc.VectorSubcoreMesh(core_axis_name="c", subcore_axis_name="s")

    @functools.partial(
        pl.kernel,
        mesh=mesh,
        compiler_params=pltpu.CompilerParams(needs_layout_passes=False),
        out_type=(
            jax.ShapeDtypeStruct((B, D), jnp.float32),
            jax.ShapeDtypeStruct((B, D), jnp.float32),
        ),
        scratch_types=[
            pltpu.VMEM((bpw,), jnp.int32),
            pltpu.VMEM((NBUF, D, 128), jnp.float32),
            pltpu.VMEM((L, D), jnp.float32),
            pltpu.SemaphoreType.DMA,
            pltpu.SemaphoreType.DMA,
        ],
    )
    def k(uid, pid, ut_t, pt_t, u_out, p_out,
          ids_v, colb_v, rows_v, sem0, sem1):
        wid = lax.axis_index("s") * NC + lax.axis_index("c")
        base = pl.multiple_of(wid * bpw, bpw)
        sems = [sem0, sem1]
        lanes = lax.iota(jnp.int32, L)
        jvecs = [lanes + q * L for q in range(D // L)]

        def one_table(idx_hbm, tab_t, out_hbm):
            pltpu.sync_copy(idx_hbm.at[pl.ds(base, bpw)], ids_v)

            def group_body(g, carry):
                vec = ids_v[pl.ds(g * L, L)]

                def fire(sub):
                    sem = sems[sub % 2]
                    for l in range(SUB):
                        i = sub * SUB + l
                        id_s = vec[i]
                        tc = lax.shift_right_logical(id_s, 7)
                        src = pl.multiple_of(tc * 128, 128)
                        pltpu.async_copy(
                            tab_t.at[:, pl.ds(src, 128)],
                            colb_v.at[i % NBUF], sem)

                def drain(sub):
                    sem = sems[sub % 2]
                    for l in range(SUB):
                        pltpu.make_async_copy(
                            tab_t.at[:, pl.ds(0, 128)],
                            colb_v.at[l % NBUF], sem).wait()

                def extract(sub):
                    drain(sub)
                    for l in range(SUB):
                        i = sub * SUB + l
                        off = jnp.full((L,), vec[i] & 127, jnp.int32)
                        buf = colb_v.at[i % NBUF]
                        for q in range(D // L):
                            rows_v[i, pl.ds(q * L, L)] = plsc.load_gather(
                                buf, [jvecs[q], off])

                fire(0)
                for sub in range(NSUB):
                    if sub + 1 < NSUB:
                        fire(sub + 1)
                    extract(sub)
                pltpu.sync_copy(rows_v, out_hbm.at[pl.ds(base + g * L, L)])
                return carry

            lax.fori_loop(0, NG, group_body, 0)

        one_table(uid, ut_t, u_out)
        one_table(pid, pt_t, p_out)

    return k


def kernel(user_ids, product_ids, user_table, product_table):
    V, D = user_table.shape
    B = user_ids.shape[0]
    k = _build(B, D, V)
    return k(user_ids.astype(jnp.int32), product_ids.astype(jnp.int32),
             user_table.T, product_table.T)


# speedup vs baseline: 2.3582x; 1.0478x over previous
import functools

import jax
import jax.numpy as jnp
from jax import lax
from jax.experimental import pallas as pl
from jax.experimental.pallas import tpu as pltpu
from jax.experimental.pallas import tpu_sc as plsc


@functools.cache
def _build(B, D, V):
    info = plsc.get_sparse_core_info()
    NC, NS, L = info.num_cores, info.num_subcores, info.num_lanes
    NW = NC * NS
    assert B % (L * NW) == 0 and D % L == 0 and V % 8 == 0
    bpw = B // NW
    NG = bpw // L
    mesh = plsc.VectorSubcoreMesh(core_axis_name="c", subcore_axis_name="s")

    @functools.partial(
        pl.kernel,
        mesh=mesh,
        out_type=(
            jax.ShapeDtypeStruct((B, D), jnp.float32),
            jax.ShapeDtypeStruct((B, D), jnp.float32),
        ),
        scratch_types=[
            pltpu.VMEM((bpw,), jnp.int32),
            pltpu.VMEM((bpw, D), jnp.float32),
            pltpu.SemaphoreType.DMA,
        ],
    )
    def k(uid, pid, ut3, pt3, u_out, p_out, ids_v, rows_v, sem):
        wid = lax.axis_index("s") * NC + lax.axis_index("c")
        base = pl.multiple_of(wid * bpw, bpw)

        def one_table(idx_hbm, tab3, out_hbm):
            pltpu.sync_copy(idx_hbm.at[pl.ds(base, bpw)], ids_v)

            def fire(g):
                vec = ids_v[pl.ds(g * L, L)]
                for l in range(L):
                    id_s = vec[l]
                    bid = lax.shift_right_logical(id_s, 3)
                    sub = id_s & 7
                    pltpu.async_copy(
                        tab3.at[bid, sub], rows_v.at[g * L + l], sem)

            def drain():
                for _ in range(L):
                    pltpu.make_async_copy(
                        tab3.at[0, 0], rows_v.at[0], sem).wait()

            fire(0)

            def group_body(g, carry):
                fire(g)
                drain()
                return carry

            lax.fori_loop(1, NG, group_body, 0)
            drain()
            pltpu.sync_copy(rows_v, out_hbm.at[pl.ds(base, bpw)])

        one_table(uid, ut3, u_out)
        one_table(pid, pt3, p_out)

    return k


def kernel(user_ids, product_ids, user_table, product_table):
    V, D = user_table.shape
    B = user_ids.shape[0]
    k = _build(B, D, V)
    ut3 = user_table.reshape(V // 8, 8, D)
    pt3 = product_table.reshape(V // 8, 8, D)
    return k(user_ids.astype(jnp.int32), product_ids.astype(jnp.int32),
             ut3, pt3)


# speedup vs baseline: 2.3916x; 1.0142x over previous
import functools

import jax
import jax.numpy as jnp
from jax import lax
from jax.experimental import pallas as pl
from jax.experimental.pallas import tpu as pltpu
from jax.experimental.pallas import tpu_sc as plsc


@functools.cache
def _build(B, D, V):
    info = plsc.get_sparse_core_info()
    NC, NS, L = info.num_cores, info.num_subcores, info.num_lanes
    NW = NC * NS
    assert B % (L * NW) == 0 and D % L == 0 and V % 8 == 0
    bpw = B // NW
    NG = bpw // L
    mesh = plsc.VectorSubcoreMesh(core_axis_name="c", subcore_axis_name="s")

    @functools.partial(
        pl.kernel,
        mesh=mesh,
        out_type=(
            jax.ShapeDtypeStruct((B, D), jnp.float32),
            jax.ShapeDtypeStruct((B, D), jnp.float32),
        ),
        scratch_types=[
            pltpu.VMEM((bpw,), jnp.int32),
            pltpu.VMEM((bpw, D), jnp.float32),
            pltpu.SemaphoreType.DMA,
        ],
    )
    def k(uid, pid, ut3, pt3, u_out, p_out, ids_v, rows_v, sem):
        wid = lax.axis_index("s") * NC + lax.axis_index("c")
        base = pl.multiple_of(wid * bpw, bpw)

        def one_table(idx_hbm, tab3, out_hbm):
            pltpu.sync_copy(idx_hbm.at[pl.ds(base, bpw)], ids_v)

            def fire(g):
                vec = ids_v[pl.ds(g * L, L)]
                for l in range(L):
                    id_s = vec[l]
                    bid = lax.shift_right_logical(id_s, 3)
                    sub = id_s & 7
                    pltpu.async_copy(
                        tab3.at[bid, sub], rows_v.at[g * L + l], sem)

            def drain():
                for _ in range(L):
                    pltpu.make_async_copy(
                        tab3.at[0, 0], rows_v.at[0], sem).wait()

            fire(0)
            fire(1)

            def group_body(g, carry):
                fire(g)
                drain()
                return carry

            lax.fori_loop(2, NG, group_body, 0)
            drain()
            drain()
            pltpu.sync_copy(rows_v, out_hbm.at[pl.ds(base, bpw)])

        one_table(uid, ut3, u_out)
        one_table(pid, pt3, p_out)

    return k


def kernel(user_ids, product_ids, user_table, product_table):
    V, D = user_table.shape
    B = user_ids.shape[0]
    k = _build(B, D, V)
    ut3 = user_table.reshape(V // 8, 8, D)
    pt3 = product_table.reshape(V // 8, 8, D)
    return k(user_ids.astype(jnp.int32), product_ids.astype(jnp.int32),
             ut3, pt3)
